# baseline (speedup 1.0000x reference)
import functools

import jax
import jax.numpy as jnp
from jax import lax
from jax.experimental import pallas as pl
from jax.experimental.pallas import tpu as pltpu
from jax.experimental.pallas import tpu_sc as plsc

_NC = 2
_NS = 16
_NW = _NC * _NS
_CHUNK = 80
_NBUF = 2

_DNUMS = lax.GatherDimensionNumbers(
    offset_dims=(), collapsed_slice_dims=(0,), start_index_map=(0,))


def _take16(x, idx):
    return lax.gather(x, idx[:, None], _DNUMS, (1,),
                      mode=lax.GatherScatterMode.PROMISE_IN_BOUNDS)


def _build(n_nodes, d_words, n_edges):
    assert n_edges % _NW == 0
    edges_per_w = n_edges // _NW
    assert edges_per_w % _CHUNK == 0
    n_chunks = edges_per_w // _CHUNK
    n_grp = _CHUNK // 16
    n_k = d_words // 16
    n_main = (n_chunks - (_NBUF + 1)) // _NBUF
    n_tail = n_chunks - n_main * _NBUF

    mesh = plsc.VectorSubcoreMesh(
        core_axis_name="c", subcore_axis_name="s",
        num_cores=_NC, num_subcores=_NS)

    @functools.partial(
        pl.kernel,
        out_type=jax.ShapeDtypeStruct((n_edges,), jnp.float32),
        mesh=mesh,
        scratch_types=[
            pltpu.VMEM((edges_per_w,), jnp.int32),
            pltpu.VMEM((edges_per_w,), jnp.int32),
            pltpu.VMEM((_NBUF, _CHUNK, d_words), jnp.int32),
            pltpu.VMEM((_NBUF, _CHUNK, d_words), jnp.int32),
            pltpu.VMEM((edges_per_w,), jnp.float32),
            [pltpu.SemaphoreType.DMA] * _NBUF,
            [pltpu.SemaphoreType.DMA] * _NBUF,
        ],
        compiler_params=pltpu.CompilerParams(use_tc_tiling_on_sc=False),
    )
    def decoder(z_hbm, src_hbm, dst_hbm, out_hbm, si_v, di_v, sr_v, dr_v,
                o_v, sems_s, sems_d):
        wid = lax.axis_index("s") * _NC + lax.axis_index("c")
        base_w = wid * edges_per_w
        lane = lax.iota(jnp.int32, 16)
        masks = [(lane & d) == 0 for d in (1, 2, 4, 8)]
        perms = [lane ^ d for d in (1, 2, 4, 8)]
        def prod2(vs, vd):
            sa = lax.bitcast_convert_type(vs << 16, jnp.float32)
            sb = lax.bitcast_convert_type(vs, jnp.float32)
            da = lax.bitcast_convert_type(vd << 16, jnp.float32)
            db = lax.bitcast_convert_type(vd, jnp.float32)
            return sa * da + sb * db

        pltpu.sync_copy(src_hbm.at[pl.ds(base_w, edges_per_w)], si_v)
        pltpu.sync_copy(dst_hbm.at[pl.ds(base_w, edges_per_w)], di_v)

        def fire(c, slot):
            pltpu.async_copy(
                z_hbm.at[si_v.at[pl.ds(c * _CHUNK, _CHUNK)]],
                sr_v.at[slot], sems_s[slot])
            pltpu.async_copy(
                z_hbm.at[di_v.at[pl.ds(c * _CHUNK, _CHUNK)]],
                dr_v.at[slot], sems_d[slot])

        def drain(slot):
            pltpu.make_async_copy(z_hbm.at[pl.ds(0, _CHUNK)],
                                  sr_v.at[slot], sems_s[slot]).wait()
            pltpu.make_async_copy(z_hbm.at[pl.ds(0, _CHUNK)],
                                  dr_v.at[slot], sems_d[slot]).wait()

        def compute(c, slot):
            sr, dr = sr_v.at[slot], dr_v.at[slot]

            @pl.loop(0, n_grp)
            def _grp(g):
                init = tuple(
                    prod2(sr[g * 16 + e, pl.ds(0, 16)],
                          dr[g * 16 + e, pl.ds(0, 16)])
                    for e in range(16))

                @pl.loop(1, n_k, init_carry=init)
                def _kstep(k, accs):
                    o = k * 16
                    return tuple(
                        accs[e] + prod2(sr[g * 16 + e, pl.ds(o, 16)],
                                        dr[g * 16 + e, pl.ds(o, 16)])
                        for e in range(16))

                vecs = list(_kstep)
                for m, p in zip(masks, perms):
                    vecs = [jnp.where(m, a, _take16(b, p))
                            + jnp.where(m, _take16(a, p), b)
                            for a, b in zip(vecs[0::2], vecs[1::2])]
                res = vecs[0]
                o_v[pl.ds(c * _CHUNK + g * 16, 16)] = 1.0 / (1.0 + jnp.exp(-res))

        for s in range(_NBUF - 1):
            fire(s, s)

        @pl.loop(0, n_main)
        def _ring(j):
            c0 = _NBUF * j
            for b in range(_NBUF):
                drain(b)
                fire(c0 + b + (_NBUF - 1), (b + (_NBUF - 1)) % _NBUF)
                compute(c0 + b, b)

        c0 = n_main * _NBUF
        for t in range(n_tail):
            c = c0 + t
            slot = c % _NBUF
            drain(slot)
            nxt = c + (_NBUF - 1)
            if nxt < n_chunks:
                fire(nxt, nxt % _NBUF)
            compute(c, slot)

        pltpu.sync_copy(o_v, out_hbm.at[pl.ds(base_w, edges_per_w)])

    return decoder


def kernel(z, edge_index):
    n_nodes, d_feat = z.shape
    n_edges = edge_index.shape[1]
    lo16 = lax.bitcast_convert_type(
        z[:, :d_feat // 2].astype(jnp.bfloat16), jnp.uint16).astype(jnp.uint32)
    tb = lax.bitcast_convert_type(z[:, d_feat // 2:], jnp.uint32)
    h = (tb - lo16 + jnp.uint32(0x8000)) >> 16
    z_packed = lax.bitcast_convert_type((h << 16) | lo16, jnp.int32)
    fn = _build(n_nodes, d_feat // 2, n_edges)
    ei = edge_index.astype(jnp.int32)
    return fn(z_packed, ei[0], ei[1])


# speedup vs baseline: 1.3590x; 1.3590x over previous
import functools

import jax
import jax.numpy as jnp
from jax import lax
from jax.experimental import pallas as pl
from jax.experimental.pallas import tpu as pltpu
from jax.experimental.pallas import tpu_sc as plsc

_NC = 2
_NS = 16
_NW = _NC * _NS
_CHUNK = 80
_NBUF = 4

_DNUMS = lax.GatherDimensionNumbers(
    offset_dims=(), collapsed_slice_dims=(0,), start_index_map=(0,))


def _take16(x, idx):
    return lax.gather(x, idx[:, None], _DNUMS, (1,),
                      mode=lax.GatherScatterMode.PROMISE_IN_BOUNDS)


def _build(n_nodes, d_words, n_edges):
    assert n_edges % _NW == 0
    edges_per_w = n_edges // _NW
    assert edges_per_w % _CHUNK == 0
    n_chunks = edges_per_w // _CHUNK
    n_grp = _CHUNK // 16
    n_k = d_words // 16
    n_main = (n_chunks - (_NBUF + 1)) // _NBUF
    n_tail = n_chunks - n_main * _NBUF

    mesh = plsc.VectorSubcoreMesh(
        core_axis_name="c", subcore_axis_name="s",
        num_cores=_NC, num_subcores=_NS)

    @functools.partial(
        pl.kernel,
        out_type=jax.ShapeDtypeStruct((n_edges,), jnp.float32),
        mesh=mesh,
        scratch_types=[
            pltpu.VMEM((edges_per_w,), jnp.int32),
            pltpu.VMEM((edges_per_w,), jnp.int32),
            pltpu.VMEM((_NBUF, _CHUNK, d_words), jnp.int32),
            pltpu.VMEM((_NBUF, _CHUNK, d_words), jnp.int32),
            pltpu.VMEM((edges_per_w,), jnp.float32),
            [pltpu.SemaphoreType.DMA] * _NBUF,
            [pltpu.SemaphoreType.DMA] * _NBUF,
        ],
        compiler_params=pltpu.CompilerParams(
            use_tc_tiling_on_sc=False,
            disable_bounds_checks=True,
            disable_semaphore_checks=True),
    )
    def decoder(z_hbm, src_hbm, dst_hbm, out_hbm, si_v, di_v, sr_v, dr_v,
                o_v, sems_s, sems_d):
        wid = lax.axis_index("s") * _NC + lax.axis_index("c")
        base_w = wid * edges_per_w
        lane = lax.iota(jnp.int32, 16)
        masks = [(lane & d) == 0 for d in (1, 2, 4, 8)]
        perms = [lane ^ d for d in (1, 2, 4, 8)]
        def prod2(vs, vd):
            sa = lax.bitcast_convert_type(vs << 16, jnp.float32)
            sb = lax.bitcast_convert_type(vs, jnp.float32)
            da = lax.bitcast_convert_type(vd << 16, jnp.float32)
            db = lax.bitcast_convert_type(vd, jnp.float32)
            return sa * da + sb * db

        pltpu.sync_copy(src_hbm.at[pl.ds(base_w, edges_per_w)], si_v)
        pltpu.sync_copy(dst_hbm.at[pl.ds(base_w, edges_per_w)], di_v)

        def fire(c, slot):
            pltpu.async_copy(
                z_hbm.at[si_v.at[pl.ds(c * _CHUNK, _CHUNK)]],
                sr_v.at[slot], sems_s[slot])
            pltpu.async_copy(
                z_hbm.at[di_v.at[pl.ds(c * _CHUNK, _CHUNK)]],
                dr_v.at[slot], sems_d[slot])

        def drain(slot):
            pltpu.make_async_copy(z_hbm.at[pl.ds(0, _CHUNK)],
                                  sr_v.at[slot], sems_s[slot]).wait()
            pltpu.make_async_copy(z_hbm.at[pl.ds(0, _CHUNK)],
                                  dr_v.at[slot], sems_d[slot]).wait()

        def compute(c, slot):
            sr, dr = sr_v.at[slot], dr_v.at[slot]

            @pl.loop(0, n_grp)
            def _grp(g):
                init = tuple(
                    prod2(sr[g * 16 + e, pl.ds(0, 16)],
                          dr[g * 16 + e, pl.ds(0, 16)])
                    for e in range(16))

                @pl.loop(1, n_k, init_carry=init)
                def _kstep(k, accs):
                    o = k * 16
                    return tuple(
                        accs[e] + prod2(sr[g * 16 + e, pl.ds(o, 16)],
                                        dr[g * 16 + e, pl.ds(o, 16)])
                        for e in range(16))

                vecs = list(_kstep)
                for m, p in zip(masks, perms):
                    vecs = [jnp.where(m, a, _take16(b, p))
                            + jnp.where(m, _take16(a, p), b)
                            for a, b in zip(vecs[0::2], vecs[1::2])]
                res = vecs[0]
                o_v[pl.ds(c * _CHUNK + g * 16, 16)] = 1.0 / (1.0 + jnp.exp(-res))

        for s in range(_NBUF - 1):
            fire(s, s)

        @pl.loop(0, n_main)
        def _ring(j):
            c0 = _NBUF * j
            for b in range(_NBUF):
                drain(b)
                fire(c0 + b + (_NBUF - 1), (b + (_NBUF - 1)) % _NBUF)
                compute(c0 + b, b)

        c0 = n_main * _NBUF
        for t in range(n_tail):
            c = c0 + t
            slot = c % _NBUF
            drain(slot)
            nxt = c + (_NBUF - 1)
            if nxt < n_chunks:
                fire(nxt, nxt % _NBUF)
            compute(c, slot)

        pltpu.sync_copy(o_v, out_hbm.at[pl.ds(base_w, edges_per_w)])

    return decoder


def kernel(z, edge_index):
    n_nodes, d_feat = z.shape
    n_edges = edge_index.shape[1]
    lo16 = lax.bitcast_convert_type(
        z[:, :d_feat // 2].astype(jnp.bfloat16), jnp.uint16).astype(jnp.uint32)
    tb = lax.bitcast_convert_type(z[:, d_feat // 2:], jnp.uint32)
    h = (tb - lo16 + jnp.uint32(0x8000)) >> 16
    z_packed = lax.bitcast_convert_type((h << 16) | lo16, jnp.int32)
    fn = _build(n_nodes, d_feat // 2, n_edges)
    ei = edge_index.astype(jnp.int32)
    return fn(z_packed, ei[0], ei[1])


# speedup vs baseline: 1.4895x; 1.0960x over previous
import functools

import jax
import jax.numpy as jnp
from jax import lax
from jax.experimental import pallas as pl
from jax.experimental.pallas import tpu as pltpu
from jax.experimental.pallas import tpu_sc as plsc

_NC = 2
_NS = 16
_NW = _NC * _NS
_CHUNK = 80
_NBUF = 4

_DNUMS = lax.GatherDimensionNumbers(
    offset_dims=(), collapsed_slice_dims=(0,), start_index_map=(0,))


def _take16(x, idx):
    return lax.gather(x, idx[:, None], _DNUMS, (1,),
                      mode=lax.GatherScatterMode.PROMISE_IN_BOUNDS)


def _build(n_nodes, d_words, n_edges):
    assert n_edges % _NW == 0
    edges_per_w = n_edges // _NW
    assert edges_per_w % _CHUNK == 0
    n_chunks = edges_per_w // _CHUNK
    n_grp = _CHUNK // 16
    n_k = d_words // 16
    n_main = (n_chunks - (_NBUF + 1)) // _NBUF
    n_tail = n_chunks - n_main * _NBUF

    mesh = plsc.VectorSubcoreMesh(
        core_axis_name="c", subcore_axis_name="s",
        num_cores=_NC, num_subcores=_NS)

    @functools.partial(
        pl.kernel,
        out_type=jax.ShapeDtypeStruct((n_edges,), jnp.float32),
        mesh=mesh,
        scratch_types=[
            pltpu.VMEM((edges_per_w,), jnp.int32),
            pltpu.VMEM((edges_per_w,), jnp.int32),
            pltpu.VMEM((_NBUF, _CHUNK, d_words), jnp.int32),
            pltpu.VMEM((_NBUF, _CHUNK, d_words), jnp.int32),
            pltpu.VMEM((edges_per_w,), jnp.float32),
            [pltpu.SemaphoreType.DMA] * _NBUF,
            [pltpu.SemaphoreType.DMA] * _NBUF,
        ],
        compiler_params=pltpu.CompilerParams(use_tc_tiling_on_sc=False),
    )
    def decoder(z_hbm, ei_hbm, out_hbm, si_v, di_v, sr_v, dr_v,
                o_v, sems_s, sems_d):
        wid = lax.axis_index("s") * _NC + lax.axis_index("c")
        base_w = wid * edges_per_w
        lane = lax.iota(jnp.int32, 16)
        masks = [(lane & d) == 0 for d in (1, 2, 4, 8)]
        perms = [lane ^ d for d in (1, 2, 4, 8)]
        def prod2(vs, vd):
            sa = lax.bitcast_convert_type(vs << 16, jnp.float32)
            sb = lax.bitcast_convert_type(vs, jnp.float32)
            da = lax.bitcast_convert_type(vd << 16, jnp.float32)
            db = lax.bitcast_convert_type(vd, jnp.float32)
            return sa * da + sb * db

        cp_si = pltpu.async_copy(
            ei_hbm.at[pl.ds(base_w, edges_per_w)], si_v, sems_s[_NBUF - 1])
        cp_di = pltpu.async_copy(
            ei_hbm.at[pl.ds(n_edges + base_w, edges_per_w)], di_v,
            sems_d[_NBUF - 1])
        cp_si.wait()
        cp_di.wait()

        def fire(c, slot):
            pltpu.async_copy(
                z_hbm.at[si_v.at[pl.ds(c * _CHUNK, _CHUNK)]],
                sr_v.at[slot], sems_s[slot])
            pltpu.async_copy(
                z_hbm.at[di_v.at[pl.ds(c * _CHUNK, _CHUNK)]],
                dr_v.at[slot], sems_d[slot])

        def drain(slot):
            pltpu.make_async_copy(z_hbm.at[pl.ds(0, _CHUNK)],
                                  sr_v.at[slot], sems_s[slot]).wait()
            pltpu.make_async_copy(z_hbm.at[pl.ds(0, _CHUNK)],
                                  dr_v.at[slot], sems_d[slot]).wait()

        def compute(c, slot):
            sr, dr = sr_v.at[slot], dr_v.at[slot]

            @pl.loop(0, n_grp)
            def _grp(g):
                init = tuple(
                    prod2(sr[g * 16 + e, pl.ds(0, 16)],
                          dr[g * 16 + e, pl.ds(0, 16)])
                    for e in range(16))

                @pl.loop(1, n_k, init_carry=init)
                def _kstep(k, accs):
                    o = k * 16
                    return tuple(
                        accs[e] + prod2(sr[g * 16 + e, pl.ds(o, 16)],
                                        dr[g * 16 + e, pl.ds(o, 16)])
                        for e in range(16))

                vecs = list(_kstep)
                for m, p in zip(masks, perms):
                    vecs = [jnp.where(m, a, _take16(b, p))
                            + jnp.where(m, _take16(a, p), b)
                            for a, b in zip(vecs[0::2], vecs[1::2])]
                res = vecs[0]
                o_v[pl.ds(c * _CHUNK + g * 16, 16)] = 1.0 / (1.0 + jnp.exp(-res))

        for s in range(_NBUF - 1):
            fire(s, s)

        @pl.loop(0, n_main)
        def _ring(j):
            c0 = _NBUF * j
            for b in range(_NBUF):
                drain(b)
                fire(c0 + b + (_NBUF - 1), (b + (_NBUF - 1)) % _NBUF)
                compute(c0 + b, b)

        c0 = n_main * _NBUF
        for t in range(n_tail):
            c = c0 + t
            slot = c % _NBUF
            drain(slot)
            nxt = c + (_NBUF - 1)
            if nxt < n_chunks:
                fire(nxt, nxt % _NBUF)
            compute(c, slot)

        pltpu.sync_copy(o_v, out_hbm.at[pl.ds(base_w, edges_per_w)])

    return decoder


def kernel(z, edge_index):
    n_nodes, d_feat = z.shape
    n_edges = edge_index.shape[1]
    lo16 = lax.bitcast_convert_type(
        z[:, :d_feat // 2].astype(jnp.bfloat16), jnp.uint16).astype(jnp.uint32)
    tb = lax.bitcast_convert_type(z[:, d_feat // 2:], jnp.uint32)
    h = (tb - lo16 + jnp.uint32(0x8000)) >> 16
    z_packed = lax.bitcast_convert_type((h << 16) | lo16, jnp.int32)
    fn = _build(n_nodes, d_feat // 2, n_edges)
    ei = edge_index.astype(jnp.int32).reshape(2 * n_edges)
    return fn(z_packed, ei)
